# baseline (speedup 1.0000x reference)
import functools

import jax
import jax.numpy as jnp
from jax import lax
from jax.experimental import pallas as pl
from jax.experimental.pallas import tpu as pltpu
from jax.experimental.pallas import tpu_sc as plsc

BATCH = 4096
HIST = 50
DIM = 128

_info = plsc.get_sparse_core_info()
NC, NS, L = _info.num_cores, _info.num_subcores, _info.num_lanes
NW = NC * NS
B = BATCH * HIST
BATCH_PER_W = BATCH // NW
NBUF = 2
NGROUPS = HIST // NBUF

_mesh = plsc.VectorSubcoreMesh(core_axis_name="c", subcore_axis_name="s")


@functools.partial(
    pl.kernel,
    mesh=_mesh,
    out_type=jax.ShapeDtypeStruct((HIST, BATCH, DIM), jnp.float32),
    scratch_types=(
        [pltpu.VMEM((HIST, BATCH_PER_W), jnp.int32)]
        + [pltpu.VMEM((BATCH_PER_W, DIM), jnp.float32)] * NBUF
        + [pltpu.SemaphoreType.DMA] * (2 * NBUF)
    ),
    compiler_params=pltpu.CompilerParams(needs_layout_passes=False),
)
def _embed_gather(xt_hbm, table_hbm, out_hbm, idx_v, *bufs):
    rows = bufs[:NBUF]
    gsem = bufs[NBUF:2 * NBUF]
    wsem = bufs[2 * NBUF:]
    wid = lax.axis_index("s") * NC + lax.axis_index("c")
    base_b = wid * BATCH_PER_W

    pltpu.sync_copy(xt_hbm.at[pl.ds(0, HIST), pl.ds(base_b, BATCH_PER_W)],
                    idx_v)

    def shift(h):
        for l in range(BATCH_PER_W // L):
            sl = pl.ds(l * L, L)
            idx_v[h, sl] = idx_v[h, sl] + 1

    def issue_gather(h, b):
        pltpu.async_copy(table_hbm.at[idx_v.at[h]], rows[b], gsem[b])

    def wait_gather(b):
        pltpu.make_async_copy(
            table_hbm.at[pl.ds(0, BATCH_PER_W)], rows[b], gsem[b]).wait()

    def issue_write(h, b):
        pltpu.async_copy(rows[b],
                         out_hbm.at[h, pl.ds(base_b, BATCH_PER_W)], wsem[b])

    def wait_write(b):
        pltpu.make_async_copy(
            rows[b], out_hbm.at[0, pl.ds(0, BATCH_PER_W)], wsem[b]).wait()

    for b in range(NBUF):
        shift(b)
    for b in range(NBUF):
        issue_gather(b, b)

    def group_body(g, _):
        for b in range(NBUF):
            shift((g + 1) * NBUF + b)
        for b in range(NBUF):
            wait_gather(b)
            issue_write(g * NBUF + b, b)
        for b in range(NBUF):
            wait_write(b)
            issue_gather((g + 1) * NBUF + b, b)
        return 0

    lax.fori_loop(0, NGROUPS - 1, group_body, 0)

    for b in range(NBUF):
        wait_gather(b)
        issue_write((NGROUPS - 1) * NBUF + b, b)
    for b in range(NBUF):
        wait_write(b)


def kernel(x, table):
    xt = jnp.transpose(x.astype(jnp.int32))
    out = _embed_gather(xt, table)
    return jnp.transpose(out, (1, 0, 2))


# speedup vs baseline: 1.0894x; 1.0894x over previous
import functools

import jax
import jax.numpy as jnp
from jax import lax
from jax.experimental import pallas as pl
from jax.experimental.pallas import tpu as pltpu
from jax.experimental.pallas import tpu_sc as plsc

BATCH = 4096
HIST = 50
DIM = 128

_info = plsc.get_sparse_core_info()
NC, NS, L = _info.num_cores, _info.num_subcores, _info.num_lanes
NW = NC * NS
B = BATCH * HIST
BATCH_PER_W = BATCH // NW
NBUF = 7
NGROUPS = (HIST - 1) // NBUF

_mesh = plsc.VectorSubcoreMesh(core_axis_name="c", subcore_axis_name="s")


@functools.partial(
    pl.kernel,
    mesh=_mesh,
    out_type=jax.ShapeDtypeStruct((HIST, BATCH, DIM), jnp.float32),
    scratch_types=(
        [pltpu.VMEM((HIST, BATCH_PER_W), jnp.int32)]
        + [pltpu.VMEM((BATCH_PER_W, DIM), jnp.float32)] * NBUF
        + [pltpu.SemaphoreType.DMA] * (2 * NBUF)
    ),
    compiler_params=pltpu.CompilerParams(needs_layout_passes=False),
)
def _embed_gather(xt_hbm, table_hbm, out_hbm, idx_v, *bufs):
    rows = bufs[:NBUF]
    gsem = bufs[NBUF:2 * NBUF]
    wsem = bufs[2 * NBUF:]
    wid = lax.axis_index("s") * NC + lax.axis_index("c")
    base_b = wid * BATCH_PER_W

    pltpu.sync_copy(xt_hbm.at[pl.ds(0, HIST), pl.ds(base_b, BATCH_PER_W)],
                    idx_v)

    def shift(h):
        for l in range(BATCH_PER_W // L):
            sl = pl.ds(l * L, L)
            idx_v[h, sl] = idx_v[h, sl] + 1

    def issue_gather(h, b):
        pltpu.async_copy(table_hbm.at[idx_v.at[h]], rows[b], gsem[b])

    def wait_gather(b):
        pltpu.make_async_copy(
            table_hbm.at[pl.ds(0, BATCH_PER_W)], rows[b], gsem[b]).wait()

    def issue_write(h, b):
        pltpu.async_copy(rows[b],
                         out_hbm.at[h, pl.ds(base_b, BATCH_PER_W)], wsem[b])

    def wait_write(b):
        pltpu.make_async_copy(
            rows[b], out_hbm.at[0, pl.ds(0, BATCH_PER_W)], wsem[b]).wait()

    for b in range(NBUF):
        shift(b)
    for b in range(NBUF):
        issue_gather(b, b)

    def group_body(g, _):
        for b in range(NBUF):
            shift((g + 1) * NBUF + b)
        for b in range(NBUF):
            wait_gather(b)
            issue_write(g * NBUF + b, b)
        for b in range(NBUF):
            wait_write(b)
            issue_gather((g + 1) * NBUF + b, b)
        return 0

    lax.fori_loop(0, NGROUPS - 1, group_body, 0)

    shift(HIST - 1)
    for b in range(NBUF):
        wait_gather(b)
        issue_write((NGROUPS - 1) * NBUF + b, b)
    wait_write(0)
    issue_gather(HIST - 1, 0)
    wait_gather(0)
    issue_write(HIST - 1, 0)
    wait_write(0)
    for b in range(1, NBUF):
        wait_write(b)


def kernel(x, table):
    xt = jnp.transpose(x.astype(jnp.int32))
    out = _embed_gather(xt, table)
    return jnp.transpose(out, (1, 0, 2))
